# baseline (speedup 1.0000x reference)
import functools

import jax
import jax.numpy as jnp
from jax import lax
from jax.experimental import pallas as pl
from jax.experimental.pallas import tpu as pltpu
from jax.experimental.pallas import tpu_sc as plsc

B = 16384
GD = 2
AD = 4
D = GD + AD
DP = 8
VOCAB = 1000
VP = 1024
TL = 128
L = 16

_info = plsc.get_sparse_core_info()
NC, NS = _info.num_cores, _info.num_subcores
NW = NC * NS
BPW = B // NW
TPW = BPW // TL
NGROUP = BPW // L

TOK_W = 2 * TL
OUT_W = DP * TL
TAB_W = DP * TL


def _body(tok_hbm, tab_hbm, out_hbm, tok_v, tab_v, out_v, sem):
    wid = lax.axis_index("s") * NC + lax.axis_index("c")
    c1 = pltpu.async_copy(
        tok_hbm.at[pl.ds(wid * (TPW * TOK_W), TPW * TOK_W)], tok_v, sem)
    c2 = pltpu.async_copy(tab_hbm, tab_v, sem)
    c1.wait()
    c2.wait()

    @pl.loop(0, NGROUP, unroll=8)
    def _group(g):
        t = g >> 3
        r = (g & 7) << 4
        toff = t * TOK_W + r
        ooff = t * OUT_W + r
        gidx = tok_v[pl.ds(toff, L)]
        aidx = tok_v[pl.ds(toff + TL, L)]
        gbase = (gidx >> 7) * TAB_W + (gidx & (TL - 1))
        abase = (aidx >> 7) * TAB_W + (aidx & (TL - 1))
        for c in range(GD):
            v = plsc.load_gather(tab_v, [gbase + (c * TL)])
            out_v[pl.ds(ooff + c * TL, L)] = v
        for c in range(AD):
            v = plsc.load_gather(tab_v, [abase + ((GD + c) * TL)])
            out_v[pl.ds(ooff + (GD + c) * TL, L)] = v

    pltpu.sync_copy(out_v, out_hbm.at[pl.ds(wid * (TPW * OUT_W), TPW * OUT_W)])


_ctx_embed = functools.partial(
    pl.kernel,
    mesh=plsc.VectorSubcoreMesh(core_axis_name="c", subcore_axis_name="s"),
    out_type=jax.ShapeDtypeStruct((B // TL * OUT_W,), jnp.float32),
    scratch_types=[
        pltpu.VMEM((TPW * TOK_W,), jnp.int32),
        pltpu.VMEM((VP // TL * TAB_W,), jnp.float32),
        pltpu.VMEM((TPW * OUT_W,), jnp.float32),
        pltpu.SemaphoreType.DMA,
    ],
    compiler_params=pltpu.CompilerParams(needs_layout_passes=False),
)(_body)


def _tiled_flat(x, rows, cols):
    return x.reshape(rows // TL, TL, cols).transpose(0, 2, 1).reshape(-1)


@jax.jit
def kernel(context_tokens, age_table, gender_table):
    tok = _tiled_flat(context_tokens.astype(jnp.int32), B, 2)
    tab = jnp.zeros((VP, DP), jnp.float32)
    tab = lax.dynamic_update_slice(
        tab, jnp.concatenate([gender_table, age_table], axis=1), (0, 0))
    flat = _ctx_embed(tok, _tiled_flat(tab, VP, DP))
    return (
        flat.reshape(B // TL, DP, TL)
        .transpose(0, 2, 1)
        .reshape(B, DP)[:, :D]
    )


# speedup vs baseline: 1.0333x; 1.0333x over previous
import functools

import jax
import jax.numpy as jnp
from jax import lax
from jax.experimental import pallas as pl
from jax.experimental.pallas import tpu as pltpu
from jax.experimental.pallas import tpu_sc as plsc

B = 16384
GD = 2
AD = 4
D = GD + AD
DP = 8
VOCAB = 1000
VP = 1024
TL = 128
L = 16

_info = plsc.get_sparse_core_info()
NC, NS = 1, _info.num_subcores
NW = NC * NS
BPW = B // NW
TPW = BPW // TL
NGROUP = BPW // L

TOK_W = 2 * TL
OUT_W = DP * TL
TAB_W = DP * TL


def _body(tok_hbm, tab_hbm, out_hbm, tok_v, tab_v, out_v, sem):
    wid = lax.axis_index("s") * NC + lax.axis_index("c")
    c1 = pltpu.async_copy(
        tok_hbm.at[pl.ds(wid * (TPW * TOK_W), TPW * TOK_W)], tok_v, sem)
    c2 = pltpu.async_copy(tab_hbm, tab_v, sem)
    c1.wait()
    c2.wait()

    @pl.loop(0, NGROUP, unroll=4)
    def _group(g):
        t = g >> 3
        r = (g & 7) << 4
        toff = t * TOK_W + r
        ooff = t * OUT_W + r
        gidx = tok_v[pl.ds(toff, L)]
        aidx = tok_v[pl.ds(toff + TL, L)]
        gbase = (gidx >> 7) * TAB_W + (gidx & (TL - 1))
        abase = (aidx >> 7) * TAB_W + (aidx & (TL - 1))
        for c in range(GD):
            v = plsc.load_gather(tab_v, [gbase + (c * TL)])
            out_v[pl.ds(ooff + c * TL, L)] = v
        for c in range(AD):
            v = plsc.load_gather(tab_v, [abase + ((GD + c) * TL)])
            out_v[pl.ds(ooff + (GD + c) * TL, L)] = v

    pltpu.sync_copy(out_v, out_hbm.at[pl.ds(wid * (TPW * OUT_W), TPW * OUT_W)])


_ctx_embed = functools.partial(
    pl.kernel,
    mesh=plsc.VectorSubcoreMesh(core_axis_name="c", subcore_axis_name="s", num_cores=1),
    out_type=jax.ShapeDtypeStruct((B // TL * OUT_W,), jnp.float32),
    scratch_types=[
        pltpu.VMEM((TPW * TOK_W,), jnp.int32),
        pltpu.VMEM((VP // TL * TAB_W,), jnp.float32),
        pltpu.VMEM((TPW * OUT_W,), jnp.float32),
        pltpu.SemaphoreType.DMA,
    ],
    compiler_params=pltpu.CompilerParams(needs_layout_passes=False),
)(_body)


def _tiled_flat(x, rows, cols):
    return x.reshape(rows // TL, TL, cols).transpose(0, 2, 1).reshape(-1)


@jax.jit
def kernel(context_tokens, age_table, gender_table):
    tok = _tiled_flat(context_tokens.astype(jnp.int32), B, 2)
    tab = jnp.zeros((VP, DP), jnp.float32)
    tab = lax.dynamic_update_slice(
        tab, jnp.concatenate([gender_table, age_table], axis=1), (0, 0))
    flat = _ctx_embed(tok, _tiled_flat(tab, VP, DP))
    return (
        flat.reshape(B // TL, DP, TL)
        .transpose(0, 2, 1)
        .reshape(B, DP)[:, :D]
    )
